# baseline (speedup 1.0000x reference)
import jax
import jax.numpy as jnp
from jax import lax
from jax.experimental import pallas as pl
from jax.experimental.pallas import tpu as pltpu
from jax.experimental.pallas import tpu_sc as plsc

NC = 2
NS = 16
NW = NC * NS
LANES = 16

SEQ = 2048
BATCH = 16
EMBED = 64

TOTAL = SEQ * BATCH
PER_W = TOTAL // NW
S_PER_W = SEQ // NW
CHUNK = 128
NCHUNK = PER_W // CHUNK
D_VREGS = EMBED // LANES


def _embed_body(table_hbm, idx_hbm, pos_hbm, out_hbm, idx_v, rows_v, pos_v, sem):
    wid = lax.axis_index("s") * NC + lax.axis_index("c")

    pltpu.sync_copy(idx_hbm.at[pl.ds(wid * NCHUNK, NCHUNK)], idx_v)

    copies = [
        pltpu.async_copy(
            table_hbm.at[idx_v.at[j]],
            rows_v.at[pl.ds(j * CHUNK, CHUNK)],
            sem,
        )
        for j in range(NCHUNK)
    ]
    pltpu.sync_copy(pos_hbm.at[pl.ds(wid * S_PER_W, S_PER_W)], pos_v)
    for c in copies:
        c.wait()

    def add_pos(i, carry):
        base = i * BATCH
        for j in range(D_VREGS):
            sl = pl.ds(j * LANES, LANES)
            p = pos_v[i, sl]
            for b in range(BATCH):
                rows_v[base + b, sl] = rows_v[base + b, sl] + p
        return carry

    lax.fori_loop(0, S_PER_W, add_pos, 0)

    pltpu.sync_copy(rows_v, out_hbm.at[pl.ds(wid * PER_W, PER_W)])


@jax.jit
def _embed(idx2, tokens_embeddings, position_embeddings):
    mesh = plsc.VectorSubcoreMesh(core_axis_name="c", subcore_axis_name="s")
    return pl.kernel(
        _embed_body,
        out_type=jax.ShapeDtypeStruct((TOTAL, EMBED), jnp.float32),
        mesh=mesh,
        compiler_params=pltpu.CompilerParams(use_tc_tiling_on_sc=False),
        scratch_types=[
            pltpu.VMEM((NCHUNK, CHUNK), jnp.int32),
            pltpu.VMEM((PER_W, EMBED), jnp.float32),
            pltpu.VMEM((S_PER_W, EMBED), jnp.float32),
            pltpu.SemaphoreType.DMA,
        ],
    )(tokens_embeddings, idx2, position_embeddings)


def kernel(x, tokens_embeddings, position_embeddings):
    idx2 = x.astype(jnp.int32).reshape(NW * NCHUNK, CHUNK)
    out = _embed(idx2, tokens_embeddings, position_embeddings)
    return out.reshape(SEQ, BATCH, EMBED)


# speedup vs baseline: 1.6762x; 1.6762x over previous
import jax
import jax.numpy as jnp
from jax import lax
from jax.experimental import pallas as pl
from jax.experimental.pallas import tpu as pltpu
from jax.experimental.pallas import tpu_sc as plsc

NC = 2
NS = 16
NW = NC * NS
LANES = 16

SEQ = 2048
BATCH = 16
EMBED = 64

TOTAL = SEQ * BATCH
PER_W = TOTAL // NW
S_PER_W = SEQ // NW
HALF = PER_W // 2
S_HALF = S_PER_W // 2
GROUPS = HALF // LANES
D_VREGS = EMBED // LANES


def _embed_body(table_hbm, idx_hbm, pos_hbm, out_hbm, idx_v, rows_v, pos_v, sem):
    wid = lax.axis_index("s") * NC + lax.axis_index("c")

    pltpu.sync_copy(idx_hbm.at[pl.ds(wid * 8, 8)], idx_v)
    pltpu.sync_copy(pos_hbm.at[pl.ds(wid * S_PER_W, S_PER_W)], pos_v)

    for h in range(2):
        def fire(g, carry):
            flat = h * HALF + g * LANES
            toks = idx_v[flat // 128, pl.ds(flat % 128, LANES)]
            for b in range(LANES):
                pltpu.async_copy(
                    table_hbm.at[pl.ds(toks[b], 1)],
                    rows_v.at[pl.ds(g * LANES + b, 1)],
                    sem,
                )
            return carry

        lax.fori_loop(0, GROUPS, fire, 0, unroll=2)

        pltpu.make_async_copy(
            table_hbm.at[pl.ds(0, HALF)], rows_v, sem
        ).wait()

        def add_pos(i, carry):
            base = i * BATCH
            for j in range(D_VREGS):
                sl = pl.ds(j * LANES, LANES)
                p = pos_v[h * S_HALF + i, sl]
                for b in range(BATCH):
                    rows_v[base + b, sl] = rows_v[base + b, sl] + p
            return carry

        lax.fori_loop(0, S_HALF, add_pos, 0)

        pltpu.sync_copy(rows_v, out_hbm.at[pl.ds(wid * PER_W + h * HALF, HALF)])


@jax.jit
def _embed(idx2, tokens_embeddings, position_embeddings):
    mesh = plsc.VectorSubcoreMesh(core_axis_name="c", subcore_axis_name="s")
    return pl.kernel(
        _embed_body,
        out_type=jax.ShapeDtypeStruct((TOTAL, EMBED), jnp.float32),
        mesh=mesh,
        scratch_types=[
            pltpu.VMEM((8, 128), jnp.int32),
            pltpu.VMEM((HALF, EMBED), jnp.float32),
            pltpu.VMEM((S_PER_W, EMBED), jnp.float32),
            pltpu.SemaphoreType.DMA,
        ],
    )(tokens_embeddings, idx2, position_embeddings)


def kernel(x, tokens_embeddings, position_embeddings):
    idx2 = x.astype(jnp.int32).reshape(NW * 8, 128)
    out = _embed(idx2, tokens_embeddings, position_embeddings)
    return out.reshape(SEQ, BATCH, EMBED)
